# baseline (speedup 1.0000x reference)
import functools

import jax
import jax.numpy as jnp
from jax import lax
from jax.experimental import pallas as pl
from jax.experimental.pallas import tpu as pltpu
from jax.experimental.pallas import tpu_sc as plsc

N_AT = 32768
N_MOLS = 16
N_ELEM = 100
DIM = 256
TBLW = 128

_NC = 2
_NS = 16
_L = 16
_NW = _NC * _NS
_CHUNK = N_AT // _NW
_NVEC = _CHUNK // _L
_UNROLL = 8


def _round_bf16_vec(x):
    u = lax.bitcast_convert_type(x, jnp.uint32)
    u = u + jnp.uint32(0x7FFF) + ((u >> jnp.uint32(16)) & jnp.uint32(1))
    u = u & jnp.uint32(0xFFFF0000)
    return lax.bitcast_convert_type(u, jnp.float32)


@functools.cache
def _make_sc_hist():
    mesh = plsc.VectorSubcoreMesh(
        core_axis_name="c", subcore_axis_name="s", num_cores=_NC
    )
    return functools.partial(
        pl.kernel,
        mesh=mesh,
        compiler_params=pltpu.CompilerParams(needs_layout_passes=False),
        out_type=jax.ShapeDtypeStruct((_NW, N_MOLS, TBLW), jnp.float32),
        scratch_types=[
            pltpu.VMEM((_CHUNK,), jnp.int32),
            pltpu.VMEM((_CHUNK,), jnp.float32),
            pltpu.VMEM((_CHUNK,), jnp.float32),
            pltpu.VMEM((_CHUNK,), jnp.float32),
            pltpu.VMEM((N_MOLS, TBLW), jnp.float32),
            pltpu.VMEM((_L * 16,), jnp.float32),
            pltpu.VMEM((_L * 16,), jnp.float32),
            pltpu.VMEM((_L * 16,), jnp.float32),
            pltpu.SemaphoreType.DMA,
        ],
    )(_sc_hist_body)


def _sc_hist_body(c_hbm, x_hbm, y_hbm, z_hbm, out_hbm,
                  c_v, x_v, y_v, z_v, acc, stx, sty, stz, sem):
    wid = lax.axis_index("s") * _NC + lax.axis_index("c")
    base = wid * _CHUNK
    cps = [
        pltpu.make_async_copy(c_hbm.at[pl.ds(base, _CHUNK)], c_v, sem),
        pltpu.make_async_copy(x_hbm.at[pl.ds(base, _CHUNK)], x_v, sem),
        pltpu.make_async_copy(y_hbm.at[pl.ds(base, _CHUNK)], y_v, sem),
        pltpu.make_async_copy(z_hbm.at[pl.ds(base, _CHUNK)], z_v, sem),
    ]
    for cp in cps:
        cp.start()

    zeros = jnp.zeros((_L,), jnp.float32)

    def zbody(i, c):
        for j in range(TBLW // _L):
            acc[i, pl.ds(j * _L, _L)] = zeros
        return c

    lax.fori_loop(0, N_MOLS, zbody, 0)

    def zbody2(i, c):
        stx[pl.ds(i * _L, _L)] = zeros
        sty[pl.ds(i * _L, _L)] = zeros
        stz[pl.ds(i * _L, _L)] = zeros
        return c

    lax.fori_loop(0, 16, zbody2, 0)

    for cp in cps:
        cp.wait()

    ones = jnp.ones((_L,), jnp.float32)
    iota = lax.iota(jnp.int32, _L)
    lane16 = iota * _L

    def step(off):
        cv = c_v[pl.ds(off, _L)]
        x = x_v[pl.ds(off, _L)]
        y = y_v[pl.ds(off, _L)]
        z = z_v[pl.ds(off, _L)]
        bv = cv >> 7
        av = cv & 127
        plsc.addupdate_scatter(acc, [bv, av], ones)
        sp = lane16 + bv
        plsc.addupdate_scatter(stx, [sp], x)
        plsc.addupdate_scatter(sty, [sp], y)
        plsc.addupdate_scatter(stz, [sp], z)

    def body(i, c):
        for u in range(4):
            step((i * 4 + u) * _L)
        return c

    lax.fori_loop(0, _NVEC // 4, body, 0)

    def fold(l, carry):
        v0, v1, v2 = carry
        o = l * _L
        v0 = v0 + stx[pl.ds(o, _L)]
        v1 = v1 + sty[pl.ds(o, _L)]
        v2 = v2 + stz[pl.ds(o, _L)]
        return (v0, v1, v2)

    v0, v1, v2 = lax.fori_loop(0, _L, fold, (zeros, zeros, zeros))
    col = jnp.full((_L,), 0, jnp.int32)
    plsc.store_scatter(acc, [iota, col + 100], v0)
    plsc.store_scatter(acc, [iota, col + 101], v1)
    plsc.store_scatter(acc, [iota, col + 102], v2)

    pltpu.sync_copy(acc, out_hbm.at[wid])


def _erf(x):
    s = jnp.sign(x)
    ax = jnp.abs(x)
    t = 1.0 / (1.0 + 0.3275911 * ax)
    poly = t * (0.254829592 + t * (-0.284496736 + t * (1.421413741
           + t * (-1.453152027 + t * 1.061405429))))
    return s * (1.0 - poly * jnp.exp(-ax * ax))


def _tc_mlp_body(part_ref, emb_ref, p_ref, w1_ref, b1_ref, w2_ref, b2_ref,
                 out_ref, wext_ref):
    wext_ref[0:N_ELEM, :] = emb_ref[...]
    wext_ref[N_ELEM:N_ELEM + 3, :] = _round_bf16_vec(p_ref[...])
    wext_ref[N_ELEM + 3:TBLW, :] = jnp.zeros(
        (TBLW - N_ELEM - 3, DIM), jnp.float32)

    s = jnp.sum(part_ref[...], axis=0)
    red = jnp.dot(s, wext_ref[...], precision=lax.Precision.HIGHEST,
                  preferred_element_type=jnp.float32)
    h = jnp.dot(red, w1_ref[...],
                preferred_element_type=jnp.float32) + b1_ref[...]
    g = h * 0.5 * (1.0 + _erf(h * 0.7071067811865476))
    out_ref[...] = (
        jnp.dot(_round_bf16_vec(g), _round_bf16_vec(w2_ref[...]),
                precision=lax.Precision.HIGHEST,
                preferred_element_type=jnp.float32) + b2_ref[...]
    )


def kernel(atomic_numbers, pos, batch, embed, P, W1, b1, W2, b2):
    a = atomic_numbers.astype(jnp.int32)
    b = batch.astype(jnp.int32)
    posf = _round_bf16_vec(pos.astype(jnp.float32))
    px, py, pz = posf[:, 0], posf[:, 1], posf[:, 2]
    comb = b * TBLW + a

    part = _make_sc_hist()(comb, px, py, pz)

    out = pl.pallas_call(
        _tc_mlp_body,
        out_shape=jax.ShapeDtypeStruct((N_MOLS, 1), jnp.float32),
        scratch_shapes=[pltpu.VMEM((TBLW, DIM), jnp.float32)],
    )(part, embed, P, W1, b1.reshape(1, DIM), W2, b2.reshape(1, 1))
    return out


# speedup vs baseline: 1.0065x; 1.0065x over previous
import functools

import jax
import jax.numpy as jnp
from jax import lax
from jax.experimental import pallas as pl
from jax.experimental.pallas import tpu as pltpu
from jax.experimental.pallas import tpu_sc as plsc

N_AT = 32768
N_MOLS = 16
N_ELEM = 100
DIM = 256
TBLW = 128

_NC = 2
_NS = 16
_L = 16
_NW = _NC * _NS
_CHUNK = N_AT // _NW
_NVEC = _CHUNK // _L
_UNROLL = 8


def _round_bf16_vec(x):
    u = lax.bitcast_convert_type(x, jnp.uint32)
    u = u + jnp.uint32(0x7FFF) + ((u >> jnp.uint32(16)) & jnp.uint32(1))
    u = u & jnp.uint32(0xFFFF0000)
    return lax.bitcast_convert_type(u, jnp.float32)


@functools.cache
def _make_sc_hist():
    mesh = plsc.VectorSubcoreMesh(
        core_axis_name="c", subcore_axis_name="s", num_cores=_NC
    )
    return functools.partial(
        pl.kernel,
        mesh=mesh,
        compiler_params=pltpu.CompilerParams(needs_layout_passes=False),
        out_type=jax.ShapeDtypeStruct((_NW, N_MOLS, TBLW), jnp.float32),
        scratch_types=[
            pltpu.VMEM((_CHUNK,), jnp.int32),
            pltpu.VMEM((_CHUNK,), jnp.float32),
            pltpu.VMEM((_CHUNK,), jnp.float32),
            pltpu.VMEM((_CHUNK,), jnp.float32),
            pltpu.VMEM((N_MOLS, TBLW), jnp.float32),
            pltpu.VMEM((_L * 16,), jnp.float32),
            pltpu.VMEM((_L * 16,), jnp.float32),
            pltpu.VMEM((_L * 16,), jnp.float32),
            pltpu.SemaphoreType.DMA,
        ],
    )(_sc_hist_body)


def _sc_hist_body(c_hbm, x_hbm, y_hbm, z_hbm, out_hbm,
                  c_v, x_v, y_v, z_v, acc, stx, sty, stz, sem):
    wid = lax.axis_index("s") * _NC + lax.axis_index("c")
    base = wid * _CHUNK
    cps = [
        pltpu.make_async_copy(c_hbm.at[pl.ds(base, _CHUNK)], c_v, sem),
        pltpu.make_async_copy(x_hbm.at[pl.ds(base, _CHUNK)], x_v, sem),
        pltpu.make_async_copy(y_hbm.at[pl.ds(base, _CHUNK)], y_v, sem),
        pltpu.make_async_copy(z_hbm.at[pl.ds(base, _CHUNK)], z_v, sem),
    ]
    for cp in cps:
        cp.start()

    zeros = jnp.zeros((_L,), jnp.float32)

    def zbody(i, c):
        for j in range(TBLW // _L):
            acc[i, pl.ds(j * _L, _L)] = zeros
        return c

    lax.fori_loop(0, N_MOLS, zbody, 0)

    def zbody2(i, c):
        stx[pl.ds(i * _L, _L)] = zeros
        sty[pl.ds(i * _L, _L)] = zeros
        stz[pl.ds(i * _L, _L)] = zeros
        return c

    lax.fori_loop(0, 16, zbody2, 0)

    for cp in cps:
        cp.wait()

    ones = jnp.ones((_L,), jnp.float32)
    iota = lax.iota(jnp.int32, _L)
    lane16 = iota * _L

    def step(off):
        cv = c_v[pl.ds(off, _L)]
        x = x_v[pl.ds(off, _L)]
        y = y_v[pl.ds(off, _L)]
        z = z_v[pl.ds(off, _L)]
        bv = cv >> 7
        av = cv & 127
        plsc.addupdate_scatter(acc, [bv, av], ones)
        sp = lane16 + bv
        plsc.addupdate_scatter(stx, [sp], x)
        plsc.addupdate_scatter(sty, [sp], y)
        plsc.addupdate_scatter(stz, [sp], z)

    def body(i, c):
        step(i * 2 * _L)
        step((i * 2 + 1) * _L)
        return c

    lax.fori_loop(0, _NVEC // 2, body, 0)

    def fold(l, carry):
        v0, v1, v2 = carry
        o = l * _L
        v0 = v0 + stx[pl.ds(o, _L)]
        v1 = v1 + sty[pl.ds(o, _L)]
        v2 = v2 + stz[pl.ds(o, _L)]
        return (v0, v1, v2)

    v0, v1, v2 = lax.fori_loop(0, _L, fold, (zeros, zeros, zeros))
    col = jnp.full((_L,), 0, jnp.int32)
    plsc.store_scatter(acc, [iota, col + 100], v0)
    plsc.store_scatter(acc, [iota, col + 101], v1)
    plsc.store_scatter(acc, [iota, col + 102], v2)

    pltpu.sync_copy(acc, out_hbm.at[wid])


def _erf(x):
    s = jnp.sign(x)
    ax = jnp.abs(x)
    t = 1.0 / (1.0 + 0.3275911 * ax)
    poly = t * (0.254829592 + t * (-0.284496736 + t * (1.421413741
           + t * (-1.453152027 + t * 1.061405429))))
    return s * (1.0 - poly * jnp.exp(-ax * ax))


def _tc_mlp_body(part_ref, emb_ref, p_ref, w1_ref, b1_ref, w2_ref, b2_ref,
                 out_ref, wext_ref):
    wext_ref[0:N_ELEM, :] = emb_ref[...]
    wext_ref[N_ELEM:N_ELEM + 3, :] = _round_bf16_vec(p_ref[...])
    wext_ref[N_ELEM + 3:TBLW, :] = jnp.zeros(
        (TBLW - N_ELEM - 3, DIM), jnp.float32)

    s = jnp.sum(part_ref[...], axis=0)
    red = jnp.dot(s, wext_ref[...], precision=lax.Precision.HIGHEST,
                  preferred_element_type=jnp.float32)
    h = jnp.dot(red, w1_ref[...],
                preferred_element_type=jnp.float32) + b1_ref[...]
    g = h * 0.5 * (1.0 + _erf(h * 0.7071067811865476))
    out_ref[...] = (
        jnp.dot(_round_bf16_vec(g), _round_bf16_vec(w2_ref[...]),
                precision=lax.Precision.HIGHEST,
                preferred_element_type=jnp.float32) + b2_ref[...]
    )


def kernel(atomic_numbers, pos, batch, embed, P, W1, b1, W2, b2):
    a = atomic_numbers.astype(jnp.int32)
    b = batch.astype(jnp.int32)
    posf = _round_bf16_vec(pos.astype(jnp.float32))
    px, py, pz = posf[:, 0], posf[:, 1], posf[:, 2]
    comb = b * TBLW + a

    part = _make_sc_hist()(comb, px, py, pz)

    out = pl.pallas_call(
        _tc_mlp_body,
        out_shape=jax.ShapeDtypeStruct((N_MOLS, 1), jnp.float32),
        scratch_shapes=[pltpu.VMEM((TBLW, DIM), jnp.float32)],
    )(part, embed, P, W1, b1.reshape(1, DIM), W2, b2.reshape(1, 1))
    return out
